# Initial kernel scaffold; baseline (speedup 1.0000x reference)
import jax
import jax.numpy as jnp
from jax.experimental import pallas as pl


def kernel(x, W, b):
    raise NotImplementedError("write your pallas kernel here")


# speedup vs baseline: 2.8299x; 2.8299x over previous
import numpy as np
import jax
import jax.numpy as jnp
from jax.experimental import pallas as pl

B_TOTAL = 16384
D = 128
BLK = 2048
SUB = BLK // 128
_TINY = np.float32(np.finfo(np.float32).tiny)

_R0 = (13, 15, 26, 6)
_R1 = (17, 29, 16, 24)


def _threefry_bits(cnt):
    k0 = jnp.uint32(0)
    k1 = jnp.uint32(42)
    k2 = k0 ^ k1 ^ jnp.uint32(0x1BD11BDA)

    def four_rounds(x0, x1, rots):
        for r in rots:
            x0 = x0 + x1
            x1 = (x1 << jnp.uint32(r)) | (x1 >> jnp.uint32(32 - r))
            x1 = x0 ^ x1
        return x0, x1

    x0 = jnp.full_like(cnt, k0)
    x1 = cnt + k1
    x0, x1 = four_rounds(x0, x1, _R0)
    x0 = x0 + k1
    x1 = x1 + (k2 + jnp.uint32(1))
    x0, x1 = four_rounds(x0, x1, _R1)
    x0 = x0 + k2
    x1 = x1 + (k0 + jnp.uint32(2))
    x0, x1 = four_rounds(x0, x1, _R0)
    x0 = x0 + k0
    x1 = x1 + (k1 + jnp.uint32(3))
    x0, x1 = four_rounds(x0, x1, _R1)
    x0 = x0 + k1
    x1 = x1 + (k2 + jnp.uint32(4))
    x0, x1 = four_rounds(x0, x1, _R0)
    x0 = x0 + k2
    x1 = x1 + (k0 + jnp.uint32(5))
    return x0 ^ x1


def _gumbel(bits):
    fb = (bits >> jnp.uint32(9)) | jnp.uint32(0x3F800000)
    f = jax.lax.bitcast_convert_type(fb, jnp.float32) - jnp.float32(1.0)
    u = jnp.maximum(_TINY, f + _TINY)
    return -jnp.log(-jnp.log(u))


def _body(x_ref, w_ref, b_ref, act_ref, lpi_ref, neg_ref, hp_ref):
    blk = pl.program_id(0)
    x = x_ref[...]
    w = w_ref[...]
    lt = jax.lax.dot_general(w, x, (((0,), (1,)), ((), ())),
                             preferred_element_type=jnp.float32)

    s_iota = jax.lax.broadcasted_iota(jnp.int32, (SUB, 128), 0)
    l_iota = jax.lax.broadcasted_iota(jnp.int32, (SUB, 128), 1)
    r3 = (s_iota * 128 + l_iota) * 3 + blk * (BLK * 3)

    lg = []
    ys = []
    for j in range(3):
        lj = jnp.reshape(lt[j:j + 1, :], (SUB, 128)) + b_ref[0, j]
        g = _gumbel(_threefry_bits((r3 + j).astype(jnp.uint32)))
        lg.append(lj)
        ys.append(g + lj)

    l0, l1, l2 = lg
    m = jnp.maximum(jnp.maximum(l0, l1), l2)
    e0 = jnp.exp(l0 - m)
    e1 = jnp.exp(l1 - m)
    e2 = jnp.exp(l2 - m)
    ls = jnp.log(e0 + e1 + e2)
    lp0 = (l0 - m) - ls
    lp1 = (l1 - m) - ls
    lp2 = (l2 - m) - ls

    y0, y1, y2 = ys
    a = jnp.where(y1 > y0, jnp.int32(1), jnp.int32(0))
    a = jnp.where(y2 > jnp.maximum(y0, y1), jnp.int32(2), a)

    lpi = jnp.where(a == 0, lp0, jnp.where(a == 1, lp1, lp2))
    hp = jnp.exp(lp1)
    neg = -jnp.log(hp)

    act_ref[...] = a
    lpi_ref[...] = lpi
    neg_ref[...] = neg
    hp_ref[...] = hp


def kernel(x, W, b):
    wp = jnp.pad(W, ((0, 0), (0, 5)))
    bp = jnp.pad(b, (0, 5)).reshape(1, 8)
    grid = (B_TOTAL // BLK,)
    out_rows = B_TOTAL // 128
    act, lpi, neg, hp = pl.pallas_call(
        _body,
        grid=grid,
        in_specs=[
            pl.BlockSpec((BLK, D), lambda i: (i, 0)),
            pl.BlockSpec((D, 8), lambda i: (0, 0)),
            pl.BlockSpec((1, 8), lambda i: (0, 0)),
        ],
        out_specs=[pl.BlockSpec((SUB, 128), lambda i: (i, 0))] * 4,
        out_shape=[
            jax.ShapeDtypeStruct((out_rows, 128), jnp.int32),
            jax.ShapeDtypeStruct((out_rows, 128), jnp.float32),
            jax.ShapeDtypeStruct((out_rows, 128), jnp.float32),
            jax.ShapeDtypeStruct((out_rows, 128), jnp.float32),
        ],
    )(x, wp, bp)
    rs = lambda t: t.reshape(B_TOTAL, 1)
    return (rs(act), rs(lpi), rs(neg), rs(hp))


# speedup vs baseline: 4.0676x; 1.4374x over previous
import numpy as np
import jax
import jax.numpy as jnp
from jax.experimental import pallas as pl

B_TOTAL = 16384
D = 128
BLK = 4096
SUB = BLK // 128
_TINY = np.float32(np.finfo(np.float32).tiny)

_R0 = (13, 15, 26, 6)
_R1 = (17, 29, 16, 24)


def _threefry_bits(cnt):
    k0 = jnp.uint32(0)
    k1 = jnp.uint32(42)
    k2 = k0 ^ k1 ^ jnp.uint32(0x1BD11BDA)

    def four_rounds(x0, x1, rots):
        for r in rots:
            x0 = x0 + x1
            x1 = (x1 << jnp.uint32(r)) | (x1 >> jnp.uint32(32 - r))
            x1 = x0 ^ x1
        return x0, x1

    x0 = jnp.full_like(cnt, k0)
    x1 = cnt + k1
    x0, x1 = four_rounds(x0, x1, _R0)
    x0 = x0 + k1
    x1 = x1 + (k2 + jnp.uint32(1))
    x0, x1 = four_rounds(x0, x1, _R1)
    x0 = x0 + k2
    x1 = x1 + (k0 + jnp.uint32(2))
    x0, x1 = four_rounds(x0, x1, _R0)
    x0 = x0 + k0
    x1 = x1 + (k1 + jnp.uint32(3))
    x0, x1 = four_rounds(x0, x1, _R1)
    x0 = x0 + k1
    x1 = x1 + (k2 + jnp.uint32(4))
    x0, x1 = four_rounds(x0, x1, _R0)
    x0 = x0 + k2
    x1 = x1 + (k0 + jnp.uint32(5))
    return x0 ^ x1


def _gumbel(bits):
    fb = (bits >> jnp.uint32(9)) | jnp.uint32(0x3F800000)
    f = jax.lax.bitcast_convert_type(fb, jnp.float32) - jnp.float32(1.0)
    u = jnp.maximum(_TINY, f + _TINY)
    return -jnp.log(-jnp.log(u))


def _body(x_ref, w_ref, b_ref, act_ref, lpi_ref, neg_ref, hp_ref):
    blk = pl.program_id(0)
    x = x_ref[...]
    w = w_ref[...]
    lt = jax.lax.dot_general(w, x, (((0,), (1,)), ((), ())),
                             preferred_element_type=jnp.float32)

    s_iota = jax.lax.broadcasted_iota(jnp.int32, (SUB, 128), 0)
    l_iota = jax.lax.broadcasted_iota(jnp.int32, (SUB, 128), 1)
    r3 = (s_iota * 128 + l_iota) * 3 + blk * (BLK * 3)

    lg = []
    ys = []
    for j in range(3):
        lj = jnp.reshape(lt[j:j + 1, :], (SUB, 128)) + b_ref[0, j]
        g = _gumbel(_threefry_bits((r3 + j).astype(jnp.uint32)))
        lg.append(lj)
        ys.append(g + lj)

    l0, l1, l2 = lg
    m = jnp.maximum(jnp.maximum(l0, l1), l2)
    e0 = jnp.exp(l0 - m)
    e1 = jnp.exp(l1 - m)
    e2 = jnp.exp(l2 - m)
    ls = jnp.log(e0 + e1 + e2)
    lp0 = (l0 - m) - ls
    lp1 = (l1 - m) - ls
    lp2 = (l2 - m) - ls

    y0, y1, y2 = ys
    a = jnp.where(y1 > y0, jnp.int32(1), jnp.int32(0))
    a = jnp.where(y2 > jnp.maximum(y0, y1), jnp.int32(2), a)

    lpi = jnp.where(a == 0, lp0, jnp.where(a == 1, lp1, lp2))
    hp = jnp.exp(lp1)
    neg = -jnp.log(hp)

    act_ref[...] = a
    lpi_ref[...] = lpi
    neg_ref[...] = neg
    hp_ref[...] = hp


def kernel(x, W, b):
    bp = b.reshape(1, 3)
    grid = (B_TOTAL // BLK,)
    out_rows = B_TOTAL // 128
    act, lpi, neg, hp = pl.pallas_call(
        _body,
        grid=grid,
        in_specs=[
            pl.BlockSpec((BLK, D), lambda i: (i, 0)),
            pl.BlockSpec((D, 3), lambda i: (0, 0)),
            pl.BlockSpec((1, 3), lambda i: (0, 0)),
        ],
        out_specs=[pl.BlockSpec((SUB, 128), lambda i: (i, 0))] * 4,
        out_shape=[
            jax.ShapeDtypeStruct((out_rows, 128), jnp.int32),
            jax.ShapeDtypeStruct((out_rows, 128), jnp.float32),
            jax.ShapeDtypeStruct((out_rows, 128), jnp.float32),
            jax.ShapeDtypeStruct((out_rows, 128), jnp.float32),
        ],
    )(x, W, bp)
    rs = lambda t: t.reshape(B_TOTAL, 1)
    return (rs(act), rs(lpi), rs(neg), rs(hp))


# speedup vs baseline: 4.5449x; 1.1173x over previous
import numpy as np
import jax
import jax.numpy as jnp
from jax.experimental import pallas as pl

B_TOTAL = 16384
D = 128
BLK = 8192
SUB = BLK // 128
_TINY = np.float32(np.finfo(np.float32).tiny)

_R0 = (13, 15, 26, 6)
_R1 = (17, 29, 16, 24)


def _threefry_bits(cnt):
    k0 = jnp.uint32(0)
    k1 = jnp.uint32(42)
    k2 = k0 ^ k1 ^ jnp.uint32(0x1BD11BDA)

    def four_rounds(x0, x1, rots):
        for r in rots:
            x0 = x0 + x1
            x1 = (x1 << jnp.uint32(r)) | (x1 >> jnp.uint32(32 - r))
            x1 = x0 ^ x1
        return x0, x1

    x0 = jnp.full_like(cnt, k0)
    x1 = cnt + k1
    x0, x1 = four_rounds(x0, x1, _R0)
    x0 = x0 + k1
    x1 = x1 + (k2 + jnp.uint32(1))
    x0, x1 = four_rounds(x0, x1, _R1)
    x0 = x0 + k2
    x1 = x1 + (k0 + jnp.uint32(2))
    x0, x1 = four_rounds(x0, x1, _R0)
    x0 = x0 + k0
    x1 = x1 + (k1 + jnp.uint32(3))
    x0, x1 = four_rounds(x0, x1, _R1)
    x0 = x0 + k1
    x1 = x1 + (k2 + jnp.uint32(4))
    x0, x1 = four_rounds(x0, x1, _R0)
    x0 = x0 + k2
    x1 = x1 + (k0 + jnp.uint32(5))
    return x0 ^ x1


def _gumbel(bits):
    fb = (bits >> jnp.uint32(9)) | jnp.uint32(0x3F800000)
    f = jax.lax.bitcast_convert_type(fb, jnp.float32) - jnp.float32(1.0)
    u = jnp.maximum(_TINY, f + _TINY)
    return -jnp.log(-jnp.log(u))


def _body(x_ref, w_ref, b_ref, act_ref, lpi_ref, neg_ref, hp_ref):
    blk = pl.program_id(0)
    x = x_ref[...]
    w = w_ref[...]
    lt = jax.lax.dot_general(w, x, (((0,), (1,)), ((), ())),
                             preferred_element_type=jnp.float32)

    s_iota = jax.lax.broadcasted_iota(jnp.int32, (SUB, 128), 0)
    l_iota = jax.lax.broadcasted_iota(jnp.int32, (SUB, 128), 1)
    r3 = (s_iota * 128 + l_iota) * 3 + blk * (BLK * 3)

    lg = []
    ys = []
    for j in range(3):
        lj = jnp.reshape(lt[j:j + 1, :], (SUB, 128)) + b_ref[0, j]
        g = _gumbel(_threefry_bits((r3 + j).astype(jnp.uint32)))
        lg.append(lj)
        ys.append(g + lj)

    l0, l1, l2 = lg
    m = jnp.maximum(jnp.maximum(l0, l1), l2)
    e0 = jnp.exp(l0 - m)
    e1 = jnp.exp(l1 - m)
    e2 = jnp.exp(l2 - m)
    ls = jnp.log(e0 + e1 + e2)
    lp0 = (l0 - m) - ls
    lp1 = (l1 - m) - ls
    lp2 = (l2 - m) - ls

    y0, y1, y2 = ys
    a = jnp.where(y1 > y0, jnp.int32(1), jnp.int32(0))
    a = jnp.where(y2 > jnp.maximum(y0, y1), jnp.int32(2), a)

    lpi = jnp.where(a == 0, lp0, jnp.where(a == 1, lp1, lp2))
    hp = jnp.exp(lp1)
    neg = -jnp.log(hp)

    act_ref[...] = a
    lpi_ref[...] = lpi
    neg_ref[...] = neg
    hp_ref[...] = hp


def kernel(x, W, b):
    bp = b.reshape(1, 3)
    grid = (B_TOTAL // BLK,)
    out_rows = B_TOTAL // 128
    act, lpi, neg, hp = pl.pallas_call(
        _body,
        grid=grid,
        in_specs=[
            pl.BlockSpec((BLK, D), lambda i: (i, 0)),
            pl.BlockSpec((D, 3), lambda i: (0, 0)),
            pl.BlockSpec((1, 3), lambda i: (0, 0)),
        ],
        out_specs=[pl.BlockSpec((SUB, 128), lambda i: (i, 0))] * 4,
        out_shape=[
            jax.ShapeDtypeStruct((out_rows, 128), jnp.int32),
            jax.ShapeDtypeStruct((out_rows, 128), jnp.float32),
            jax.ShapeDtypeStruct((out_rows, 128), jnp.float32),
            jax.ShapeDtypeStruct((out_rows, 128), jnp.float32),
        ],
    )(x, W, bp)
    rs = lambda t: t.reshape(B_TOTAL, 1)
    return (rs(act), rs(lpi), rs(neg), rs(hp))
